# Initial kernel scaffold; baseline (speedup 1.0000x reference)
import jax
import jax.numpy as jnp
from jax.experimental import pallas as pl


def kernel(X, codebook, return_dist):
    raise NotImplementedError("write your pallas kernel here")


# speedup vs baseline: 1.6235x; 1.6235x over previous
import functools

import jax
import jax.numpy as jnp
from jax.experimental import pallas as pl
from jax.experimental.pallas import tpu as pltpu

_BN = 512
_BK = 1024


def _dist_argmin_kernel(x_ref, ct_ref, idx_ref, dist_ref, min_sc, arg_sc):
    k = pl.program_id(1)
    nk = pl.num_programs(1)

    @pl.when(k == 0)
    def _init():
        min_sc[...] = jnp.full(min_sc.shape, jnp.inf, jnp.float32)
        arg_sc[...] = jnp.zeros(arg_sc.shape, jnp.int32)

    x = x_ref[...]
    ct = ct_ref[...]
    dot = jnp.dot(x, ct, preferred_element_type=jnp.float32)
    x2 = jnp.sum(x * x, axis=1, keepdims=True)
    c2 = jnp.sum(ct * ct, axis=0, keepdims=True)
    d2 = (x2 + c2) - 2.0 * dot

    bk = d2.shape[1]
    bmin = jnp.min(d2, axis=1, keepdims=True)
    iota = jax.lax.broadcasted_iota(jnp.int32, d2.shape, 1)
    barg = jnp.min(jnp.where(d2 == bmin, iota, bk), axis=1, keepdims=True)
    barg = barg + k * bk

    improved = bmin < min_sc[...]
    arg_sc[...] = jnp.where(improved, barg, arg_sc[...])
    min_sc[...] = jnp.where(improved, bmin, min_sc[...])

    @pl.when(k == nk - 1)
    def _write():
        idx_ref[...] = arg_sc[...]
        dist_ref[...] = jnp.sqrt(jnp.maximum(min_sc[...], 0.0))


@functools.partial(jax.jit, static_argnames=())
def _assign(X, ct):
    n, d = X.shape
    kk = ct.shape[1]
    bn, bk = _BN, _BK
    grid = (n // bn, kk // bk)
    idx2, dist2 = pl.pallas_call(
        _dist_argmin_kernel,
        grid=grid,
        in_specs=[
            pl.BlockSpec((bn, d), lambda i, k: (i, 0)),
            pl.BlockSpec((d, bk), lambda i, k: (0, k)),
        ],
        out_specs=[
            pl.BlockSpec((bn, 1), lambda i, k: (i, 0)),
            pl.BlockSpec((bn, 1), lambda i, k: (i, 0)),
        ],
        out_shape=[
            jax.ShapeDtypeStruct((n, 1), jnp.int32),
            jax.ShapeDtypeStruct((n, 1), jnp.float32),
        ],
        scratch_shapes=[
            pltpu.VMEM((bn, 1), jnp.float32),
            pltpu.VMEM((bn, 1), jnp.int32),
        ],
        compiler_params=pltpu.CompilerParams(
            dimension_semantics=("parallel", "arbitrary"),
        ),
    )(X, ct)
    return idx2[:, 0], dist2[:, 0]


def kernel(X, codebook, return_dist):
    idx, dist = _assign(X, codebook.T)
    dist = dist * jnp.asarray(return_dist, dist.dtype)
    return (idx, dist)


# speedup vs baseline: 2.1088x; 1.2990x over previous
import functools

import jax
import jax.numpy as jnp
from jax.experimental import pallas as pl
from jax.experimental.pallas import tpu as pltpu

_BN = 1024
_BK = 1024


def _dist_argmin_kernel(x_ref, ct_ref, idx_ref, dist_ref, max_sc, arg_sc):
    k = pl.program_id(1)
    nk = pl.num_programs(1)

    @pl.when(k == 0)
    def _init():
        max_sc[...] = jnp.full(max_sc.shape, -jnp.inf, jnp.float32)
        arg_sc[...] = jnp.zeros(arg_sc.shape, jnp.int32)

    x = x_ref[...]
    ct = ct_ref[...]
    dot = jnp.dot(x, ct, preferred_element_type=jnp.float32)
    half_c2 = 0.5 * jnp.sum(ct * ct, axis=0, keepdims=True)
    score = dot - half_c2

    bk = score.shape[1]
    bmax = jnp.max(score, axis=1, keepdims=True)
    iota = jax.lax.broadcasted_iota(jnp.int32, score.shape, 1)
    barg = jnp.min(jnp.where(score == bmax, iota, bk), axis=1, keepdims=True)
    barg = barg + k * bk

    improved = bmax > max_sc[...]
    arg_sc[...] = jnp.where(improved, barg, arg_sc[...])
    max_sc[...] = jnp.where(improved, bmax, max_sc[...])

    @pl.when(k == nk - 1)
    def _write():
        x2 = jnp.sum(x * x, axis=1, keepdims=True)
        idx_ref[...] = arg_sc[...]
        dist_ref[...] = jnp.sqrt(jnp.maximum(x2 - 2.0 * max_sc[...], 0.0))


@functools.partial(jax.jit, static_argnames=())
def _assign(X, ct):
    n, d = X.shape
    kk = ct.shape[1]
    bn, bk = _BN, _BK
    grid = (n // bn, kk // bk)
    idx2, dist2 = pl.pallas_call(
        _dist_argmin_kernel,
        grid=grid,
        in_specs=[
            pl.BlockSpec((bn, d), lambda i, k: (i, 0)),
            pl.BlockSpec((d, bk), lambda i, k: (0, k)),
        ],
        out_specs=[
            pl.BlockSpec((bn, 1), lambda i, k: (i, 0)),
            pl.BlockSpec((bn, 1), lambda i, k: (i, 0)),
        ],
        out_shape=[
            jax.ShapeDtypeStruct((n, 1), jnp.int32),
            jax.ShapeDtypeStruct((n, 1), jnp.float32),
        ],
        scratch_shapes=[
            pltpu.VMEM((bn, 1), jnp.float32),
            pltpu.VMEM((bn, 1), jnp.int32),
        ],
        compiler_params=pltpu.CompilerParams(
            dimension_semantics=("parallel", "arbitrary"),
        ),
    )(X, ct)
    return idx2[:, 0], dist2[:, 0]


def kernel(X, codebook, return_dist):
    idx, dist = _assign(X, codebook.T)
    dist = dist * jnp.asarray(return_dist, dist.dtype)
    return (idx, dist)
